# baseline (speedup 1.0000x reference)
import jax
import jax.numpy as jnp
from jax import lax
from jax.experimental import pallas as pl
from jax.experimental.pallas import tpu as pltpu

E = 64
TOPK = 8
TB = 2048
NB = 8


def _router_body(x_ref, wt_ref, logits_ref, probs_ref, topw_ref, topi_ref,
                 lg_s, s_s):
    i = pl.program_id(0)

    @pl.when(i > 0)
    def _selection():
        logits = lg_s[...]
        s = s_s[...]
        eidx = lax.broadcasted_iota(jnp.int32, (TB, E), 1)
        kidx = lax.broadcasted_iota(jnp.int32, (TB, TOPK), 1)
        topi = jnp.zeros((TB, TOPK), jnp.int32)
        work = logits
        for k in range(TOPK):
            mi = jnp.argmax(work, axis=-1).reshape(TB, 1)
            topi = jnp.where(kidx == k, mi, topi)
            work = jnp.where(eidx == mi, -jnp.inf, work)
        topi_ref[...] = topi
        topl = jnp.take_along_axis(logits, topi, axis=-1)
        m = topl[:, 0:1]
        topv = jnp.exp(topl - m) / s
        ssum = jnp.sum(topv, axis=-1, keepdims=True) + 1e-6
        topw_ref[...] = topv / ssum

    @pl.when(i < NB)
    def _gate():
        xb = x_ref[...]
        wt = wt_ref[...]
        logits = jnp.dot(xb, wt, preferred_element_type=jnp.float32)
        logits_ref[...] = logits
        m = jnp.max(logits, axis=-1, keepdims=True)
        ex = jnp.exp(logits - m)
        s = jnp.sum(ex, axis=-1, keepdims=True)
        probs_ref[...] = ex / s
        lg_s[...] = logits
        s_s[...] = s


def kernel(x, W):
    b, s, h = x.shape
    n = b * s
    x2 = x.reshape(n, h)
    wt = W.T

    grid = (NB + 1,)
    out_shapes = (
        jax.ShapeDtypeStruct((n, E), jnp.float32),
        jax.ShapeDtypeStruct((n, E), jnp.float32),
        jax.ShapeDtypeStruct((n, TOPK), jnp.float32),
        jax.ShapeDtypeStruct((n, TOPK), jnp.int32),
    )
    logits, probs, topw, topi = pl.pallas_call(
        _router_body,
        grid=grid,
        in_specs=[
            pl.BlockSpec((TB, h), lambda i: (jnp.minimum(i, NB - 1), 0)),
            pl.BlockSpec((h, E), lambda i: (0, 0)),
        ],
        out_specs=(
            pl.BlockSpec((TB, E), lambda i: (jnp.minimum(i, NB - 1), 0)),
            pl.BlockSpec((TB, E), lambda i: (jnp.minimum(i, NB - 1), 0)),
            pl.BlockSpec((TB, TOPK), lambda i: (jnp.maximum(i - 1, 0), 0)),
            pl.BlockSpec((TB, TOPK), lambda i: (jnp.maximum(i - 1, 0), 0)),
        ),
        out_shape=out_shapes,
        scratch_shapes=[
            pltpu.VMEM((TB, E), jnp.float32),
            pltpu.VMEM((TB, 1), jnp.float32),
        ],
        compiler_params=pltpu.CompilerParams(
            dimension_semantics=("arbitrary",),
        ),
    )(x2, wt)

    routing_weights = topw.reshape(b, s, TOPK)
    expert_indices = topi.reshape(b, s, TOPK)
    aux = jnp.array(0.0, dtype=x.dtype)
    return (routing_weights, expert_indices, logits, probs, aux)


# speedup vs baseline: 1.0041x; 1.0041x over previous
import jax
import jax.numpy as jnp
from jax import lax
from jax.experimental import pallas as pl
from jax.experimental.pallas import tpu as pltpu

E = 64
TOPK = 8
TB = 2048


def _router_body(x_ref, wt_ref, logits_ref, probs_ref, topw_ref, topi_ref):
    xb = x_ref[...]
    wt = wt_ref[...]
    logits = jnp.dot(xb, wt, preferred_element_type=jnp.float32)
    logits_ref[...] = logits

    m = jnp.max(logits, axis=-1, keepdims=True)
    ex = jnp.exp(logits - m)
    s = jnp.sum(ex, axis=-1, keepdims=True)
    p = ex / s
    probs_ref[...] = p

    eidx = lax.broadcasted_iota(jnp.int32, (TB, E), 1)
    kidx = lax.broadcasted_iota(jnp.int32, (TB, TOPK), 1)
    topi = jnp.zeros((TB, TOPK), jnp.int32)
    work = logits
    for k in range(TOPK):
        mi = jnp.argmax(work, axis=-1).reshape(TB, 1)
        topi = jnp.where(kidx == k, mi, topi)
        work = jnp.where(eidx == mi, -jnp.inf, work)
    topi_ref[...] = topi
    topl = jnp.take_along_axis(logits, topi, axis=-1)
    topv = jnp.exp(topl - m) / s
    ssum = jnp.sum(topv, axis=-1, keepdims=True) + 1e-6
    topw_ref[...] = topv / ssum


def kernel(x, W):
    b, s, h = x.shape
    n = b * s
    x2 = x.reshape(n, h)
    wt = W.T

    grid = (n // TB,)
    out_shapes = (
        jax.ShapeDtypeStruct((n, E), jnp.float32),
        jax.ShapeDtypeStruct((n, E), jnp.float32),
        jax.ShapeDtypeStruct((n, TOPK), jnp.float32),
        jax.ShapeDtypeStruct((n, TOPK), jnp.int32),
    )
    logits, probs, topw, topi = pl.pallas_call(
        _router_body,
        grid=grid,
        in_specs=[
            pl.BlockSpec((TB, h), lambda i: (i, 0)),
            pl.BlockSpec((h, E), lambda i: (0, 0)),
        ],
        out_specs=(
            pl.BlockSpec((TB, E), lambda i: (i, 0)),
            pl.BlockSpec((TB, E), lambda i: (i, 0)),
            pl.BlockSpec((TB, TOPK), lambda i: (i, 0)),
            pl.BlockSpec((TB, TOPK), lambda i: (i, 0)),
        ),
        out_shape=out_shapes,
        compiler_params=pltpu.CompilerParams(
            dimension_semantics=("parallel",),
        ),
    )(x2, wt)

    routing_weights = topw.reshape(b, s, TOPK)
    expert_indices = topi.reshape(b, s, TOPK)
    aux = jnp.array(0.0, dtype=x.dtype)
    return (routing_weights, expert_indices, logits, probs, aux)
